# baseline (speedup 1.0000x reference)
import functools

import jax
import jax.numpy as jnp
import numpy as np
from jax.experimental import pallas as pl
from jax.experimental.pallas import tpu as pltpu

X_DIM = 3
PE_DIM = 10
HIDDEN = 64
N_LAYERS = 4
NUM_ATOMS = 16
NUM_RES = 20
CUT2 = 1.5 * 1.5
INV_NORM = 1.0 / 100.0
N_MOL = 1024
N_PRO = 2048
BATCH = 16
N = N_MOL + N_PRO
TI = 32
TJ = 64
NTI = N // TI
NTJ = N // TJ
NTI_MOL = N_MOL // TI
NTJ_MOL = N_MOL // TJ


def _silu(x):
    return x * jax.nn.sigmoid(x)


def _dot(a, b):
    return jax.lax.dot_general(a, b, (((1,), (0,)), ((), ())),
                               preferred_element_type=jnp.float32)


def _mega_kernel(
    count_ref, work_ref,
    zf_mol_ref, zf_pro_ref, xs_ref, xsti_ref, xstj_ref, idxs_ref, t_ref, pos_ref, freq_ref,
    wa1_ref, ba1_ref, wa2_ref, ba2_ref,
    wr1_ref, br1_ref, wr2_ref, br2_ref,
    weh_ref, wes_ref, wec_ref, wet_ref, wex_ref, be_ref,
    ws_ref, wd_ref, ce_ref, w2_ref, b2_ref,
    wh_ref, wa_ref, bn1_ref, wn2_ref, bn2_ref,
    wod_ref, bod_ref, woh_ref, boh_ref,
    wad1_ref, bad1_ref, wad2_ref, bad2_ref,
    wrd1_ref, brd1_ref, wrd2_ref, brd2_ref,
    pmd_ref, pmh_ref, ppd_ref, pph_ref,
    out_mol_ref, out_pro_ref,
    h_ref, a_ref, b_ref, agg_ref,
):
    hm = _dot(_silu(_dot(zf_mol_ref[...], wa1_ref[...]) + ba1_ref[...]),
              wa2_ref[...]) + ba2_ref[...]
    hp = _dot(_silu(_dot(zf_pro_ref[...], wr1_ref[...]) + br1_ref[...]),
              wr2_ref[...]) + br2_ref[...]

    idxs = idxs_ref[...]
    iot = jax.lax.broadcasted_iota(jnp.int32, (N, BATCH), 1)
    oh = (iot == idxs[:, None]).astype(jnp.float32)
    tvals = jnp.sum(oh * t_ref[...].reshape(1, BATCH), axis=1)

    base = tvals[:, None] * wet_ref[...].reshape(1, HIDDEN) + be_ref[...]
    for k in range(X_DIM):
        base = base + xs_ref[k][:, None] * wex_ref[k][None, :]

    ang = pos_ref[...][:, None] * freq_ref[...][None, :]
    pe_part = _dot(jnp.sin(ang), wes_ref[...]) + _dot(jnp.cos(ang), wec_ref[...])

    h_ref[:N_MOL, :] = base[:N_MOL, :] + _dot(hm, weh_ref[...]) + pe_part
    h_ref[N_MOL:, :] = base[N_MOL:, :] + _dot(hp, weh_ref[...])

    nwork = count_ref[0]

    for l in range(N_LAYERS):
        h = h_ref[...]
        a_ref[...] = _dot(h, ws_ref[l])
        b_ref[...] = _dot(h, wd_ref[l])
        agg_ref[...] = jnp.zeros((N, HIDDEN), jnp.float32)
        w2 = w2_ref[l]
        b2 = b2_ref[l]

        def entry(code, _w2=w2, _b2=b2, _l=l):
            ti = jax.lax.rem(code, 128)
            tj1 = jax.lax.rem(jax.lax.div(code, 128), 64)
            tj2 = jax.lax.rem(jax.lax.div(code, 8192), 64)
            et1 = jax.lax.rem(jax.lax.div(code, 524288), 4)
            et2 = jax.lax.rem(jax.lax.div(code, 2097152), 4)
            flag2 = jax.lax.rem(jax.lax.div(code, 8388608), 2)
            valida = jax.lax.div(code, 16777216)
            i0 = ti * TI
            ai = a_ref[pl.ds(i0, TI), :]
            acat = jnp.concatenate([ai, ai], axis=1)
            bj1 = b_ref[pl.ds(tj1 * TJ, TJ), :]
            bj2 = b_ref[pl.ds(tj2 * TJ, TJ), :]
            bcat = jnp.concatenate([bj1, bj2], axis=1)
            ce1 = ce_ref[_l, pl.ds(et1, 1), :]
            ce2 = ce_ref[_l, pl.ds(et2, 1), :]
            cecat = jnp.concatenate([ce1, ce2], axis=1)
            pre = acat[:, None, :] + bcat[None, :, :] + cecat[None, :, :]
            m = _silu(_dot(_silu(pre.reshape(TI * TJ, 2 * HIDDEN)), _w2) + _b2)
            xi = xsti_ref[pl.ds(ti, 1)].reshape(8, TI)
            xj1 = xstj_ref[pl.ds(tj1, 1)].reshape(8, TJ)
            xj2 = xstj_ref[pl.ds(tj2, 1)].reshape(8, TJ)
            d2a = jnp.sum((xi[:, :, None] - xj1[:, None, :]) ** 2, axis=0)
            d2b = jnp.sum((xi[:, :, None] - xj2[:, None, :]) ** 2, axis=0)
            ma = jnp.where(d2a <= CUT2, valida.astype(jnp.float32), 0.0)[:, :, None]
            mb = jnp.where(d2b <= CUT2, (flag2 * valida).astype(jnp.float32), 0.0)[:, :, None]
            lane = jax.lax.broadcasted_iota(jnp.int32, (TI, TJ, 2 * HIDDEN), 2)
            mcat = jnp.where(lane < HIDDEN,
                             jnp.broadcast_to(ma, (TI, TJ, 2 * HIDDEN)),
                             jnp.broadcast_to(mb, (TI, TJ, 2 * HIDDEN)))
            m3 = m.reshape(TI, TJ, 2 * HIDDEN) * mcat
            contrib = jnp.sum(m3, axis=1)
            return i0, contrib[:, :HIDDEN] + contrib[:, HIDDEN:]

        def tile_body(tt, carry):
            i0a, ca = entry(work_ref[3 * tt])
            i0b, cb = entry(work_ref[3 * tt + 1])
            i0c, cc = entry(work_ref[3 * tt + 2])
            agg_ref[pl.ds(i0a, TI), :] += ca
            agg_ref[pl.ds(i0b, TI), :] += cb
            agg_ref[pl.ds(i0c, TI), :] += cc
            return carry

        jax.lax.fori_loop(0, (nwork + 2) // 3, tile_body, 0)

        ag = agg_ref[...] * INV_NORM
        pre_n = _dot(h, wh_ref[l]) + _dot(ag, wa_ref[l]) + bn1_ref[l]
        h_ref[...] = h + _dot(_silu(pre_n), wn2_ref[l]) + bn2_ref[l]

    h = h_ref[...]
    disp = _dot(h, wod_ref[...]) + bod_ref[...]
    hn = _dot(h, woh_ref[...]) + boh_ref[...]
    hm_out = _dot(_silu(_dot(hn[:N_MOL, :], wad1_ref[...]) + bad1_ref[...]),
                  wad2_ref[...]) + bad2_ref[...]
    hp_out = _dot(_silu(_dot(hn[N_MOL:, :], wrd1_ref[...]) + brd1_ref[...]),
                  wrd2_ref[...]) + brd2_ref[...]
    out_mol_ref[...] = _dot(disp[:N_MOL, :], pmd_ref[...]) + _dot(hm_out, pmh_ref[...])
    out_pro_ref[...] = _dot(disp[N_MOL:, :], ppd_ref[...]) + _dot(hp_out, pph_ref[...])


def kernel(z_t_mol, z_t_pro, t, molecule_idx, protein_pocket_idx, molecule_pos, params):
    f32 = jnp.float32
    zf_mol = z_t_mol[:, X_DIM:]
    zf_pro = z_t_pro[:, X_DIM:]
    xs = jnp.zeros((8, N), f32)
    xs = xs.at[:X_DIM, :N_MOL].set(z_t_mol[:, :X_DIM].T)
    xs = xs.at[:X_DIM, N_MOL:].set(z_t_pro[:, :X_DIM].T)
    idxs = jnp.concatenate([molecule_idx.astype(jnp.int32),
                            protein_pocket_idx.astype(jnp.int32)], 0)
    xaug = xs.at[X_DIM, :].set(1000.0 * idxs.astype(f32))
    xsti = jnp.transpose(xaug.reshape(8, NTI, TI), (1, 0, 2))
    xstj = jnp.transpose(xaug.reshape(8, NTJ, TJ), (1, 0, 2))
    freq = (1.0 / (10000.0 ** (2.0 * jnp.arange(PE_DIM // 2, dtype=f32) / PE_DIM)))
    pos = molecule_pos.astype(f32)

    segi = idxs.reshape(NTI, TI)
    imin, imax = segi.min(axis=1), segi.max(axis=1)
    segj = idxs.reshape(NTJ, TJ)
    jmin, jmax = segj.min(axis=1), segj.max(axis=1)
    act = (imin[:, None] <= jmax[None, :]) & (jmin[None, :] <= imax[:, None])
    is_mol_i = jnp.arange(NTI) < NTI_MOL
    is_mol_j = jnp.arange(NTJ) < NTJ_MOL
    et = jnp.where(is_mol_i[:, None] & is_mol_j[None, :], 1,
                   jnp.where((~is_mol_i[:, None]) & (~is_mol_j[None, :]), 2, 0))
    order = jnp.argsort(jnp.where(act, 0, 1), axis=1, stable=True)
    cnt = act.sum(axis=1)
    NS = NTJ // 2 + 1
    s2 = 2 * jnp.arange(NS)
    j1 = jnp.take_along_axis(order, jnp.minimum(s2, NTJ - 1)[None, :], axis=1)
    j2 = jnp.take_along_axis(order, jnp.minimum(s2 + 1, NTJ - 1)[None, :], axis=1)
    valid1 = s2[None, :] < cnt[:, None]
    valid2 = (s2 + 1)[None, :] < cnt[:, None]
    j2 = jnp.where(valid2, j2, j1)
    ii = jnp.broadcast_to(jnp.arange(NTI)[:, None], (NTI, NS))
    et1 = et[ii, j1]
    et2 = et[ii, j2]
    codes = (ii + 128 * j1 + 8192 * j2 + 524288 * et1 + 2097152 * et2
             + 8388608 * valid2.astype(jnp.int32)
             + 16777216).astype(jnp.int32)
    sel = jnp.nonzero(valid1.ravel(), size=NTI * NS, fill_value=0)[0]
    count = valid1.sum().astype(jnp.int32)
    work = jnp.where(jnp.arange(NTI * NS, dtype=jnp.int32) < count,
                     codes.ravel()[sel], 0).astype(jnp.int32)
    count = count.reshape(1)

    p = params
    gnn = p['gnn']
    we = gnn['emb']['W']
    weh = we[X_DIM:X_DIM + HIDDEN]
    wes = we[X_DIM + HIDDEN:X_DIM + HIDDEN + PE_DIM // 2]
    wec = we[X_DIM + HIDDEN + PE_DIM // 2:X_DIM + HIDDEN + PE_DIM]
    wet = we[X_DIM + HIDDEN + PE_DIM]
    wex = we[:X_DIM]
    be = gnn['emb']['b']

    ws_l, wd_l, ce_l, w2_l, b2_l = [], [], [], [], []
    wh_l, wa_l, bn1_l, wn2_l, bn2_l = [], [], [], [], []
    for lp in gnn['layers']:
        w1 = lp['edge_mlp']['l1']['W']
        b1 = lp['edge_mlp']['l1']['b']
        ws_l.append(w1[:HIDDEN])
        wd_l.append(w1[HIDDEN:2 * HIDDEN])
        ce_l.append(p['edge_emb'] @ w1[2 * HIDDEN:] + b1[None, :])
        w2 = lp['edge_mlp']['l2']['W']
        w2blk = jnp.zeros((2 * HIDDEN, 2 * HIDDEN), jnp.float32)
        w2blk = w2blk.at[:HIDDEN, :HIDDEN].set(w2).at[HIDDEN:, HIDDEN:].set(w2)
        w2_l.append(w2blk)
        b2_l.append(jnp.concatenate([lp['edge_mlp']['l2']['b']] * 2))
        wn1 = lp['node_mlp']['l1']['W']
        wh_l.append(wn1[:HIDDEN])
        wa_l.append(wn1[HIDDEN:])
        bn1_l.append(lp['node_mlp']['l1']['b'])
        wn2_l.append(lp['node_mlp']['l2']['W'])
        bn2_l.append(lp['node_mlp']['l2']['b'])
    stack = lambda xs_: jnp.stack(xs_, 0)

    wo = gnn['out']['W']
    bo = gnn['out']['b']
    wod, bod = wo[:, :X_DIM], bo[:X_DIM]
    woh, boh = wo[:, X_DIM:X_DIM + HIDDEN], bo[X_DIM:X_DIM + HIDDEN]

    pmd = jnp.zeros((X_DIM, X_DIM + NUM_ATOMS), f32).at[:, :X_DIM].set(jnp.eye(X_DIM))
    pmh = jnp.zeros((NUM_ATOMS, X_DIM + NUM_ATOMS), f32).at[:, X_DIM:].set(jnp.eye(NUM_ATOMS))
    ppd = jnp.zeros((X_DIM, X_DIM + NUM_RES), f32).at[:, :X_DIM].set(jnp.eye(X_DIM))
    pph = jnp.zeros((NUM_RES, X_DIM + NUM_RES), f32).at[:, X_DIM:].set(jnp.eye(NUM_RES))

    smem = pl.BlockSpec(memory_space=pltpu.SMEM)
    n_vec_inputs = 49

    out = pl.pallas_call(
        _mega_kernel,
        out_shape=[
            jax.ShapeDtypeStruct((N_MOL, X_DIM + NUM_ATOMS), f32),
            jax.ShapeDtypeStruct((N_PRO, X_DIM + NUM_RES), f32),
        ],
        in_specs=[smem, smem] + [pl.BlockSpec(memory_space=pltpu.VMEM)] * n_vec_inputs,
        out_specs=[pl.BlockSpec(memory_space=pltpu.VMEM)] * 2,
        scratch_shapes=[
            pltpu.VMEM((N, HIDDEN), f32),
            pltpu.VMEM((N, HIDDEN), f32),
            pltpu.VMEM((N, HIDDEN), f32),
            pltpu.VMEM((N, HIDDEN), f32),
        ],
    )(
        count, work,
        zf_mol, zf_pro, xs, xsti, xstj, idxs, t.reshape(BATCH), pos, freq,
        p['atom_enc']['l1']['W'], p['atom_enc']['l1']['b'],
        p['atom_enc']['l2']['W'], p['atom_enc']['l2']['b'],
        p['res_enc']['l1']['W'], p['res_enc']['l1']['b'],
        p['res_enc']['l2']['W'], p['res_enc']['l2']['b'],
        weh, wes, wec, wet, wex, be,
        stack(ws_l), stack(wd_l), stack(ce_l), stack(w2_l), stack(b2_l),
        stack(wh_l), stack(wa_l), stack(bn1_l), stack(wn2_l), stack(bn2_l),
        wod, bod, woh, boh,
        p['atom_dec']['l1']['W'], p['atom_dec']['l1']['b'],
        p['atom_dec']['l2']['W'], p['atom_dec']['l2']['b'],
        p['res_dec']['l1']['W'], p['res_dec']['l1']['b'],
        p['res_dec']['l2']['W'], p['res_dec']['l2']['b'],
        pmd, pmh, ppd, pph,
    )
    return (out[0], out[1])


# speedup vs baseline: 1.0199x; 1.0199x over previous
import functools

import jax
import jax.numpy as jnp
import numpy as np
from jax.experimental import pallas as pl
from jax.experimental.pallas import tpu as pltpu

X_DIM = 3
PE_DIM = 10
HIDDEN = 64
N_LAYERS = 4
NUM_ATOMS = 16
NUM_RES = 20
CUT2 = 1.5 * 1.5
INV_NORM = 1.0 / 100.0
N_MOL = 1024
N_PRO = 2048
BATCH = 16
N = N_MOL + N_PRO
TI = 32
TJ = 64
NTI = N // TI
NTJ = N // TJ
NTI_MOL = N_MOL // TI
NTJ_MOL = N_MOL // TJ


def _silu(x):
    return x * jax.nn.sigmoid(x)


def _dot(a, b):
    return jax.lax.dot_general(a, b, (((1,), (0,)), ((), ())),
                               preferred_element_type=jnp.float32)


def _mega_kernel(
    count_ref, work_ref,
    zf_mol_ref, zf_pro_ref, xs_ref, xsti_ref, xstj_ref, idxs_ref, t_ref, pos_ref, freq_ref,
    wa1_ref, ba1_ref, wa2_ref, ba2_ref,
    wr1_ref, br1_ref, wr2_ref, br2_ref,
    weh_ref, wes_ref, wec_ref, wet_ref, wex_ref, be_ref,
    ws_ref, wd_ref, ce_ref, w2_ref, b2_ref,
    wh_ref, wa_ref, bn1_ref, wn2_ref, bn2_ref,
    wod_ref, bod_ref, woh_ref, boh_ref,
    wad1_ref, bad1_ref, wad2_ref, bad2_ref,
    wrd1_ref, brd1_ref, wrd2_ref, brd2_ref,
    pmd_ref, pmh_ref, ppd_ref, pph_ref,
    out_mol_ref, out_pro_ref,
    h_ref, a_ref, b_ref, agg_ref,
):
    hm = _dot(_silu(_dot(zf_mol_ref[...], wa1_ref[...]) + ba1_ref[...]),
              wa2_ref[...]) + ba2_ref[...]
    hp = _dot(_silu(_dot(zf_pro_ref[...], wr1_ref[...]) + br1_ref[...]),
              wr2_ref[...]) + br2_ref[...]

    idxs = idxs_ref[...]
    iot = jax.lax.broadcasted_iota(jnp.int32, (N, BATCH), 1)
    oh = (iot == idxs[:, None]).astype(jnp.float32)
    tvals = jnp.sum(oh * t_ref[...].reshape(1, BATCH), axis=1)

    base = tvals[:, None] * wet_ref[...].reshape(1, HIDDEN) + be_ref[...]
    for k in range(X_DIM):
        base = base + xs_ref[k][:, None] * wex_ref[k][None, :]

    ang = pos_ref[...][:, None] * freq_ref[...][None, :]
    pe_part = _dot(jnp.sin(ang), wes_ref[...]) + _dot(jnp.cos(ang), wec_ref[...])

    h_ref[:N_MOL, :] = base[:N_MOL, :] + _dot(hm, weh_ref[...]) + pe_part
    h_ref[N_MOL:, :] = base[N_MOL:, :] + _dot(hp, weh_ref[...])

    nwork = count_ref[0]

    for l in range(N_LAYERS):
        h = h_ref[...]
        a_ref[...] = _dot(h, ws_ref[l])
        b_ref[...] = _dot(h, wd_ref[l])
        agg_ref[...] = jnp.zeros((N, HIDDEN), jnp.float32)
        w2 = w2_ref[l]
        b2 = b2_ref[l]

        def entry(code, _w2=w2, _b2=b2, _l=l):
            ti = jax.lax.rem(code, 128)
            tj1 = jax.lax.rem(jax.lax.div(code, 128), 64)
            tj2 = jax.lax.rem(jax.lax.div(code, 8192), 64)
            et1 = jax.lax.rem(jax.lax.div(code, 524288), 4)
            et2 = jax.lax.rem(jax.lax.div(code, 2097152), 4)
            flag2 = jax.lax.rem(jax.lax.div(code, 8388608), 2)
            valida = jax.lax.div(code, 16777216)
            i0 = ti * TI
            ai = a_ref[pl.ds(i0, TI), :]
            acat = jnp.concatenate([ai, ai], axis=1)
            bj1 = b_ref[pl.ds(tj1 * TJ, TJ), :]
            bj2 = b_ref[pl.ds(tj2 * TJ, TJ), :]
            bcat = jnp.concatenate([bj1, bj2], axis=1)
            ce1 = ce_ref[_l, pl.ds(et1, 1), :]
            ce2 = ce_ref[_l, pl.ds(et2, 1), :]
            cecat = jnp.concatenate([ce1, ce2], axis=1)
            pre = acat[:, None, :] + bcat[None, :, :] + cecat[None, :, :]
            m = _silu(_dot(_silu(pre.reshape(TI * TJ, 2 * HIDDEN)), _w2) + _b2)
            xi = xsti_ref[pl.ds(ti, 1)].reshape(8, TI)
            xj1 = xstj_ref[pl.ds(tj1, 1)].reshape(8, TJ)
            xj2 = xstj_ref[pl.ds(tj2, 1)].reshape(8, TJ)
            d2a = jnp.sum((xi[:, :, None] - xj1[:, None, :]) ** 2, axis=0)
            d2b = jnp.sum((xi[:, :, None] - xj2[:, None, :]) ** 2, axis=0)
            ma = jnp.where(d2a <= CUT2, valida.astype(jnp.float32), 0.0)[:, :, None]
            mb = jnp.where(d2b <= CUT2, (flag2 * valida).astype(jnp.float32), 0.0)[:, :, None]
            lane = jax.lax.broadcasted_iota(jnp.int32, (TI, TJ, 2 * HIDDEN), 2)
            mcat = jnp.where(lane < HIDDEN,
                             jnp.broadcast_to(ma, (TI, TJ, 2 * HIDDEN)),
                             jnp.broadcast_to(mb, (TI, TJ, 2 * HIDDEN)))
            m3 = m.reshape(TI, TJ, 2 * HIDDEN) * mcat
            contrib = jnp.sum(m3, axis=1)
            return i0, contrib[:, :HIDDEN] + contrib[:, HIDDEN:]

        def tile_body(tt, carry):
            i0a, ca = entry(work_ref[2 * tt])
            i0b, cb = entry(work_ref[2 * tt + 1])
            agg_ref[pl.ds(i0a, TI), :] += ca
            agg_ref[pl.ds(i0b, TI), :] += cb
            return carry

        jax.lax.fori_loop(0, (nwork + 1) // 2, tile_body, 0)

        ag = agg_ref[...] * INV_NORM
        pre_n = _dot(h, wh_ref[l]) + _dot(ag, wa_ref[l]) + bn1_ref[l]
        h_ref[...] = h + _dot(_silu(pre_n), wn2_ref[l]) + bn2_ref[l]

    h = h_ref[...]
    disp = _dot(h, wod_ref[...]) + bod_ref[...]
    hn = _dot(h, woh_ref[...]) + boh_ref[...]
    hm_out = _dot(_silu(_dot(hn[:N_MOL, :], wad1_ref[...]) + bad1_ref[...]),
                  wad2_ref[...]) + bad2_ref[...]
    hp_out = _dot(_silu(_dot(hn[N_MOL:, :], wrd1_ref[...]) + brd1_ref[...]),
                  wrd2_ref[...]) + brd2_ref[...]
    out_mol_ref[...] = _dot(disp[:N_MOL, :], pmd_ref[...]) + _dot(hm_out, pmh_ref[...])
    out_pro_ref[...] = _dot(disp[N_MOL:, :], ppd_ref[...]) + _dot(hp_out, pph_ref[...])


def kernel(z_t_mol, z_t_pro, t, molecule_idx, protein_pocket_idx, molecule_pos, params):
    f32 = jnp.float32
    zf_mol = z_t_mol[:, X_DIM:]
    zf_pro = z_t_pro[:, X_DIM:]
    xs = jnp.zeros((8, N), f32)
    xs = xs.at[:X_DIM, :N_MOL].set(z_t_mol[:, :X_DIM].T)
    xs = xs.at[:X_DIM, N_MOL:].set(z_t_pro[:, :X_DIM].T)
    idxs = jnp.concatenate([molecule_idx.astype(jnp.int32),
                            protein_pocket_idx.astype(jnp.int32)], 0)
    xaug = xs.at[X_DIM, :].set(1000.0 * idxs.astype(f32))
    xsti = jnp.transpose(xaug.reshape(8, NTI, TI), (1, 0, 2))
    xstj = jnp.transpose(xaug.reshape(8, NTJ, TJ), (1, 0, 2))
    freq = (1.0 / (10000.0 ** (2.0 * jnp.arange(PE_DIM // 2, dtype=f32) / PE_DIM)))
    pos = molecule_pos.astype(f32)

    segi = idxs.reshape(NTI, TI)
    imin, imax = segi.min(axis=1), segi.max(axis=1)
    segj = idxs.reshape(NTJ, TJ)
    jmin, jmax = segj.min(axis=1), segj.max(axis=1)
    act = (imin[:, None] <= jmax[None, :]) & (jmin[None, :] <= imax[:, None])
    is_mol_i = jnp.arange(NTI) < NTI_MOL
    is_mol_j = jnp.arange(NTJ) < NTJ_MOL
    et = jnp.where(is_mol_i[:, None] & is_mol_j[None, :], 1,
                   jnp.where((~is_mol_i[:, None]) & (~is_mol_j[None, :]), 2, 0))
    order = jnp.argsort(jnp.where(act, 0, 1), axis=1, stable=True)
    cnt = act.sum(axis=1)
    NS = NTJ // 2 + 1
    s2 = 2 * jnp.arange(NS)
    j1 = jnp.take_along_axis(order, jnp.minimum(s2, NTJ - 1)[None, :], axis=1)
    j2 = jnp.take_along_axis(order, jnp.minimum(s2 + 1, NTJ - 1)[None, :], axis=1)
    valid1 = s2[None, :] < cnt[:, None]
    valid2 = (s2 + 1)[None, :] < cnt[:, None]
    j2 = jnp.where(valid2, j2, j1)
    ii = jnp.broadcast_to(jnp.arange(NTI)[:, None], (NTI, NS))
    et1 = et[ii, j1]
    et2 = et[ii, j2]
    codes = (ii + 128 * j1 + 8192 * j2 + 524288 * et1 + 2097152 * et2
             + 8388608 * valid2.astype(jnp.int32)
             + 16777216).astype(jnp.int32)
    sel = jnp.nonzero(valid1.ravel(), size=NTI * NS, fill_value=0)[0]
    count = valid1.sum().astype(jnp.int32)
    work = jnp.where(jnp.arange(NTI * NS, dtype=jnp.int32) < count,
                     codes.ravel()[sel], 0).astype(jnp.int32)
    count = count.reshape(1)

    p = params
    gnn = p['gnn']
    we = gnn['emb']['W']
    weh = we[X_DIM:X_DIM + HIDDEN]
    wes = we[X_DIM + HIDDEN:X_DIM + HIDDEN + PE_DIM // 2]
    wec = we[X_DIM + HIDDEN + PE_DIM // 2:X_DIM + HIDDEN + PE_DIM]
    wet = we[X_DIM + HIDDEN + PE_DIM]
    wex = we[:X_DIM]
    be = gnn['emb']['b']

    ws_l, wd_l, ce_l, w2_l, b2_l = [], [], [], [], []
    wh_l, wa_l, bn1_l, wn2_l, bn2_l = [], [], [], [], []
    for lp in gnn['layers']:
        w1 = lp['edge_mlp']['l1']['W']
        b1 = lp['edge_mlp']['l1']['b']
        ws_l.append(w1[:HIDDEN])
        wd_l.append(w1[HIDDEN:2 * HIDDEN])
        ce_l.append(p['edge_emb'] @ w1[2 * HIDDEN:] + b1[None, :])
        w2 = lp['edge_mlp']['l2']['W']
        w2blk = jnp.zeros((2 * HIDDEN, 2 * HIDDEN), jnp.float32)
        w2blk = w2blk.at[:HIDDEN, :HIDDEN].set(w2).at[HIDDEN:, HIDDEN:].set(w2)
        w2_l.append(w2blk)
        b2_l.append(jnp.concatenate([lp['edge_mlp']['l2']['b']] * 2))
        wn1 = lp['node_mlp']['l1']['W']
        wh_l.append(wn1[:HIDDEN])
        wa_l.append(wn1[HIDDEN:])
        bn1_l.append(lp['node_mlp']['l1']['b'])
        wn2_l.append(lp['node_mlp']['l2']['W'])
        bn2_l.append(lp['node_mlp']['l2']['b'])
    stack = lambda xs_: jnp.stack(xs_, 0)

    wo = gnn['out']['W']
    bo = gnn['out']['b']
    wod, bod = wo[:, :X_DIM], bo[:X_DIM]
    woh, boh = wo[:, X_DIM:X_DIM + HIDDEN], bo[X_DIM:X_DIM + HIDDEN]

    pmd = jnp.zeros((X_DIM, X_DIM + NUM_ATOMS), f32).at[:, :X_DIM].set(jnp.eye(X_DIM))
    pmh = jnp.zeros((NUM_ATOMS, X_DIM + NUM_ATOMS), f32).at[:, X_DIM:].set(jnp.eye(NUM_ATOMS))
    ppd = jnp.zeros((X_DIM, X_DIM + NUM_RES), f32).at[:, :X_DIM].set(jnp.eye(X_DIM))
    pph = jnp.zeros((NUM_RES, X_DIM + NUM_RES), f32).at[:, X_DIM:].set(jnp.eye(NUM_RES))

    smem = pl.BlockSpec(memory_space=pltpu.SMEM)
    n_vec_inputs = 49

    out = pl.pallas_call(
        _mega_kernel,
        out_shape=[
            jax.ShapeDtypeStruct((N_MOL, X_DIM + NUM_ATOMS), f32),
            jax.ShapeDtypeStruct((N_PRO, X_DIM + NUM_RES), f32),
        ],
        in_specs=[smem, smem] + [pl.BlockSpec(memory_space=pltpu.VMEM)] * n_vec_inputs,
        out_specs=[pl.BlockSpec(memory_space=pltpu.VMEM)] * 2,
        scratch_shapes=[
            pltpu.VMEM((N, HIDDEN), f32),
            pltpu.VMEM((N, HIDDEN), f32),
            pltpu.VMEM((N, HIDDEN), f32),
            pltpu.VMEM((N, HIDDEN), f32),
        ],
    )(
        count, work,
        zf_mol, zf_pro, xs, xsti, xstj, idxs, t.reshape(BATCH), pos, freq,
        p['atom_enc']['l1']['W'], p['atom_enc']['l1']['b'],
        p['atom_enc']['l2']['W'], p['atom_enc']['l2']['b'],
        p['res_enc']['l1']['W'], p['res_enc']['l1']['b'],
        p['res_enc']['l2']['W'], p['res_enc']['l2']['b'],
        weh, wes, wec, wet, wex, be,
        stack(ws_l), stack(wd_l), stack(ce_l), stack(w2_l), stack(b2_l),
        stack(wh_l), stack(wa_l), stack(bn1_l), stack(wn2_l), stack(bn2_l),
        wod, bod, woh, boh,
        p['atom_dec']['l1']['W'], p['atom_dec']['l1']['b'],
        p['atom_dec']['l2']['W'], p['atom_dec']['l2']['b'],
        p['res_dec']['l1']['W'], p['res_dec']['l1']['b'],
        p['res_dec']['l2']['W'], p['res_dec']['l2']['b'],
        pmd, pmh, ppd, pph,
    )
    return (out[0], out[1])


# speedup vs baseline: 1.0237x; 1.0038x over previous
import jax
import jax.numpy as jnp
from jax.experimental import pallas as pl
from jax.experimental.pallas import tpu as pltpu

X_DIM = 3
PE_DIM = 10
HIDDEN = 64
N_LAYERS = 4
NUM_ATOMS = 16
NUM_RES = 20
CUT2 = 1.5 * 1.5
INV_NORM = 1.0 / 100.0
N_MOL = 1024
N_PRO = 2048
BATCH = 16
N = N_MOL + N_PRO
TI = 32
TJ = 64
NTI = N // TI
NTJ = N // TJ
NTI_MOL = N_MOL // TI
NTJ_MOL = N_MOL // TJ


def _silu(x):
    return x * jax.nn.sigmoid(x)


def _dot(a, b):
    return jax.lax.dot_general(a, b, (((1,), (0,)), ((), ())),
                               preferred_element_type=jnp.float32)


def _mega_kernel(
    count_ref, work_ref,
    zf_mol_ref, zf_pro_ref, xs_ref, xsti_ref, xstj_ref, idxs_ref, t_ref, pos_ref, freq_ref,
    wa1_ref, ba1_ref, wa2_ref, ba2_ref,
    wr1_ref, br1_ref, wr2_ref, br2_ref,
    weh_ref, wes_ref, wec_ref, wet_ref, wex_ref, be_ref,
    ws_ref, wd_ref, ce_ref, w2_ref, b2_ref,
    wh_ref, wa_ref, bn1_ref, wn2_ref, bn2_ref,
    wod_ref, bod_ref, woh_ref, boh_ref,
    wad1_ref, bad1_ref, wad2_ref, bad2_ref,
    wrd1_ref, brd1_ref, wrd2_ref, brd2_ref,
    pmd_ref, pmh_ref, ppd_ref, pph_ref,
    out_mol_ref, out_pro_ref,
    h_ref, a_ref, b_ref, agg_ref,
):
    hm = _dot(_silu(_dot(zf_mol_ref[...], wa1_ref[...]) + ba1_ref[...]),
              wa2_ref[...]) + ba2_ref[...]
    hp = _dot(_silu(_dot(zf_pro_ref[...], wr1_ref[...]) + br1_ref[...]),
              wr2_ref[...]) + br2_ref[...]

    idxs = idxs_ref[...]
    iot = jax.lax.broadcasted_iota(jnp.int32, (N, BATCH), 1)
    oh = (iot == idxs[:, None]).astype(jnp.float32)
    tvals = jnp.sum(oh * t_ref[...].reshape(1, BATCH), axis=1)

    base = tvals[:, None] * wet_ref[...].reshape(1, HIDDEN) + be_ref[...]
    for k in range(X_DIM):
        base = base + xs_ref[k][:, None] * wex_ref[k][None, :]

    ang = pos_ref[...][:, None] * freq_ref[...][None, :]
    pe_part = _dot(jnp.sin(ang), wes_ref[...]) + _dot(jnp.cos(ang), wec_ref[...])

    h_ref[:N_MOL, :] = base[:N_MOL, :] + _dot(hm, weh_ref[...]) + pe_part
    h_ref[N_MOL:, :] = base[N_MOL:, :] + _dot(hp, weh_ref[...])

    nwork = count_ref[0]

    for l in range(N_LAYERS):
        h = h_ref[...]
        a_ref[...] = _dot(h, ws_ref[l])
        b_ref[...] = _dot(h, wd_ref[l])
        agg_ref[...] = jnp.zeros((N, HIDDEN), jnp.float32)
        w2 = w2_ref[l]
        b2 = b2_ref[l]

        def entry(code, _w2=w2, _b2=b2, _l=l):
            ti = jax.lax.rem(code, 128)
            tj1 = jax.lax.rem(jax.lax.div(code, 128), 64)
            tj2 = jax.lax.rem(jax.lax.div(code, 8192), 64)
            et1 = jax.lax.rem(jax.lax.div(code, 524288), 4)
            et2 = jax.lax.rem(jax.lax.div(code, 2097152), 4)
            flag2 = jax.lax.rem(jax.lax.div(code, 8388608), 2)
            valida = jax.lax.div(code, 16777216)
            i0 = ti * TI
            ai = a_ref[pl.ds(i0, TI), :]
            acat = jnp.concatenate([ai, ai], axis=1)
            bj1 = b_ref[pl.ds(tj1 * TJ, TJ), :]
            bj2 = b_ref[pl.ds(tj2 * TJ, TJ), :]
            bcat = jnp.concatenate([bj1, bj2], axis=1)
            ce1 = ce_ref[_l, pl.ds(et1, 1), :]
            ce2 = ce_ref[_l, pl.ds(et2, 1), :]
            cecat = jnp.concatenate([ce1, ce2], axis=1)
            pre = acat[:, None, :] + bcat[None, :, :] + cecat[None, :, :]
            m = _silu(_dot(_silu(pre.reshape(TI * TJ, 2 * HIDDEN)), _w2) + _b2)
            xi = xsti_ref[pl.ds(ti, 1)].reshape(8, TI)
            xj1 = xstj_ref[pl.ds(tj1, 1)].reshape(8, TJ)
            xj2 = xstj_ref[pl.ds(tj2, 1)].reshape(8, TJ)
            d2a = jnp.sum((xi[:, :, None] - xj1[:, None, :]) ** 2, axis=0)
            d2b = jnp.sum((xi[:, :, None] - xj2[:, None, :]) ** 2, axis=0)
            ma = jnp.where(d2a <= CUT2, valida.astype(jnp.float32), 0.0)[:, :, None]
            mb = jnp.where(d2b <= CUT2, (flag2 * valida).astype(jnp.float32), 0.0)[:, :, None]
            lane = jax.lax.broadcasted_iota(jnp.int32, (TI, TJ, 2 * HIDDEN), 2)
            mcat = jnp.where(lane < HIDDEN,
                             jnp.broadcast_to(ma, (TI, TJ, 2 * HIDDEN)),
                             jnp.broadcast_to(mb, (TI, TJ, 2 * HIDDEN)))
            m3 = m.reshape(TI, TJ, 2 * HIDDEN) * mcat
            contrib = jnp.sum(m3, axis=1)
            return i0, contrib[:, :HIDDEN] + contrib[:, HIDDEN:]

        def tile_body(tt, carry):
            i0a, ca = entry(work_ref[2 * tt])
            i0b, cb = entry(work_ref[2 * tt + 1])
            agg_ref[pl.ds(i0a, TI), :] += ca
            agg_ref[pl.ds(i0b, TI), :] += cb
            return carry

        jax.lax.fori_loop(0, (nwork + 1) // 2, tile_body, 0)

        ag = agg_ref[...] * INV_NORM
        pre_n = _dot(h, wh_ref[l]) + _dot(ag, wa_ref[l]) + bn1_ref[l]
        h_ref[...] = h + _dot(_silu(pre_n), wn2_ref[l]) + bn2_ref[l]

    h = h_ref[...]
    disp = _dot(h, wod_ref[...]) + bod_ref[...]
    hn = _dot(h, woh_ref[...]) + boh_ref[...]
    hm_out = _dot(_silu(_dot(hn[:N_MOL, :], wad1_ref[...]) + bad1_ref[...]),
                  wad2_ref[...]) + bad2_ref[...]
    hp_out = _dot(_silu(_dot(hn[N_MOL:, :], wrd1_ref[...]) + brd1_ref[...]),
                  wrd2_ref[...]) + brd2_ref[...]
    out_mol_ref[...] = _dot(disp[:N_MOL, :], pmd_ref[...]) + _dot(hm_out, pmh_ref[...])
    out_pro_ref[...] = _dot(disp[N_MOL:, :], ppd_ref[...]) + _dot(hp_out, pph_ref[...])


def kernel(z_t_mol, z_t_pro, t, molecule_idx, protein_pocket_idx, molecule_pos, params):
    f32 = jnp.float32
    zf_mol = z_t_mol[:, X_DIM:]
    zf_pro = z_t_pro[:, X_DIM:]
    xs = jnp.zeros((8, N), f32)
    xs = xs.at[:X_DIM, :N_MOL].set(z_t_mol[:, :X_DIM].T)
    xs = xs.at[:X_DIM, N_MOL:].set(z_t_pro[:, :X_DIM].T)
    idxs = jnp.concatenate([molecule_idx.astype(jnp.int32),
                            protein_pocket_idx.astype(jnp.int32)], 0)
    xaug = xs.at[X_DIM, :].set(1000.0 * idxs.astype(f32))
    xsti = jnp.transpose(xaug.reshape(8, NTI, TI), (1, 0, 2))
    xstj = jnp.transpose(xaug.reshape(8, NTJ, TJ), (1, 0, 2))
    freq = (1.0 / (10000.0 ** (2.0 * jnp.arange(PE_DIM // 2, dtype=f32) / PE_DIM)))
    pos = molecule_pos.astype(f32)

    segi = idxs.reshape(NTI, TI)
    imin, imax = segi.min(axis=1), segi.max(axis=1)
    segj = idxs.reshape(NTJ, TJ)
    jmin, jmax = segj.min(axis=1), segj.max(axis=1)
    act = (imin[:, None] <= jmax[None, :]) & (jmin[None, :] <= imax[:, None])
    is_mol_i = jnp.arange(NTI) < NTI_MOL
    is_mol_j = jnp.arange(NTJ) < NTJ_MOL
    et = jnp.where(is_mol_i[:, None] & is_mol_j[None, :], 1,
                   jnp.where((~is_mol_i[:, None]) & (~is_mol_j[None, :]), 2, 0))
    order = jnp.argsort(jnp.where(act, 0, 1), axis=1, stable=True)
    cnt = act.sum(axis=1)
    NS = NTJ // 2 + 1
    s2 = 2 * jnp.arange(NS)
    j1 = jnp.take_along_axis(order, jnp.minimum(s2, NTJ - 1)[None, :], axis=1)
    j2 = jnp.take_along_axis(order, jnp.minimum(s2 + 1, NTJ - 1)[None, :], axis=1)
    valid1 = s2[None, :] < cnt[:, None]
    valid2 = (s2 + 1)[None, :] < cnt[:, None]
    j2 = jnp.where(valid2, j2, j1)
    ii = jnp.broadcast_to(jnp.arange(NTI)[:, None], (NTI, NS))
    et1 = et[ii, j1]
    et2 = et[ii, j2]
    codes = (ii + 128 * j1 + 8192 * j2 + 524288 * et1 + 2097152 * et2
             + 8388608 * valid2.astype(jnp.int32)
             + 16777216).astype(jnp.int32)
    sel = jnp.nonzero(valid1.ravel(), size=NTI * NS, fill_value=0)[0]
    count = valid1.sum().astype(jnp.int32)
    work = jnp.where(jnp.arange(NTI * NS, dtype=jnp.int32) < count,
                     codes.ravel()[sel], 0).astype(jnp.int32)
    count = count.reshape(1)

    p = params
    gnn = p['gnn']
    we = gnn['emb']['W']
    weh = we[X_DIM:X_DIM + HIDDEN]
    wes = we[X_DIM + HIDDEN:X_DIM + HIDDEN + PE_DIM // 2]
    wec = we[X_DIM + HIDDEN + PE_DIM // 2:X_DIM + HIDDEN + PE_DIM]
    wet = we[X_DIM + HIDDEN + PE_DIM]
    wex = we[:X_DIM]
    be = gnn['emb']['b']

    ws_l, wd_l, ce_l, w2_l, b2_l = [], [], [], [], []
    wh_l, wa_l, bn1_l, wn2_l, bn2_l = [], [], [], [], []
    for lp in gnn['layers']:
        w1 = lp['edge_mlp']['l1']['W']
        b1 = lp['edge_mlp']['l1']['b']
        ws_l.append(w1[:HIDDEN])
        wd_l.append(w1[HIDDEN:2 * HIDDEN])
        ce_l.append(p['edge_emb'] @ w1[2 * HIDDEN:] + b1[None, :])
        w2 = lp['edge_mlp']['l2']['W']
        w2blk = jnp.zeros((2 * HIDDEN, 2 * HIDDEN), jnp.float32)
        w2blk = w2blk.at[:HIDDEN, :HIDDEN].set(w2).at[HIDDEN:, HIDDEN:].set(w2)
        w2_l.append(w2blk)
        b2_l.append(jnp.concatenate([lp['edge_mlp']['l2']['b']] * 2))
        wn1 = lp['node_mlp']['l1']['W']
        wh_l.append(wn1[:HIDDEN])
        wa_l.append(wn1[HIDDEN:])
        bn1_l.append(lp['node_mlp']['l1']['b'])
        wn2_l.append(lp['node_mlp']['l2']['W'])
        bn2_l.append(lp['node_mlp']['l2']['b'])
    stack = lambda xs_: jnp.stack(xs_, 0)

    wo = gnn['out']['W']
    bo = gnn['out']['b']
    wod, bod = wo[:, :X_DIM], bo[:X_DIM]
    woh, boh = wo[:, X_DIM:X_DIM + HIDDEN], bo[X_DIM:X_DIM + HIDDEN]

    pmd = jnp.zeros((X_DIM, X_DIM + NUM_ATOMS), f32).at[:, :X_DIM].set(jnp.eye(X_DIM))
    pmh = jnp.zeros((NUM_ATOMS, X_DIM + NUM_ATOMS), f32).at[:, X_DIM:].set(jnp.eye(NUM_ATOMS))
    ppd = jnp.zeros((X_DIM, X_DIM + NUM_RES), f32).at[:, :X_DIM].set(jnp.eye(X_DIM))
    pph = jnp.zeros((NUM_RES, X_DIM + NUM_RES), f32).at[:, X_DIM:].set(jnp.eye(NUM_RES))

    smem = pl.BlockSpec(memory_space=pltpu.SMEM)
    n_vec_inputs = 49

    out = pl.pallas_call(
        _mega_kernel,
        out_shape=[
            jax.ShapeDtypeStruct((N_MOL, X_DIM + NUM_ATOMS), f32),
            jax.ShapeDtypeStruct((N_PRO, X_DIM + NUM_RES), f32),
        ],
        in_specs=[smem, smem] + [pl.BlockSpec(memory_space=pltpu.VMEM)] * n_vec_inputs,
        out_specs=[pl.BlockSpec(memory_space=pltpu.VMEM)] * 2,
        scratch_shapes=[
            pltpu.VMEM((N, HIDDEN), f32),
            pltpu.VMEM((N, HIDDEN), f32),
            pltpu.VMEM((N, HIDDEN), f32),
            pltpu.VMEM((N, HIDDEN), f32),
        ],
    )(
        count, work,
        zf_mol, zf_pro, xs, xsti, xstj, idxs, t.reshape(BATCH), pos, freq,
        p['atom_enc']['l1']['W'], p['atom_enc']['l1']['b'],
        p['atom_enc']['l2']['W'], p['atom_enc']['l2']['b'],
        p['res_enc']['l1']['W'], p['res_enc']['l1']['b'],
        p['res_enc']['l2']['W'], p['res_enc']['l2']['b'],
        weh, wes, wec, wet, wex, be,
        stack(ws_l), stack(wd_l), stack(ce_l), stack(w2_l), stack(b2_l),
        stack(wh_l), stack(wa_l), stack(bn1_l), stack(wn2_l), stack(bn2_l),
        wod, bod, woh, boh,
        p['atom_dec']['l1']['W'], p['atom_dec']['l1']['b'],
        p['atom_dec']['l2']['W'], p['atom_dec']['l2']['b'],
        p['res_dec']['l1']['W'], p['res_dec']['l1']['b'],
        p['res_dec']['l2']['W'], p['res_dec']['l2']['b'],
        pmd, pmh, ppd, pph,
    )
    return (out[0], out[1])
